# Initial kernel scaffold; baseline (speedup 1.0000x reference)
import functools

import jax
import jax.numpy as jnp
from jax import lax
from jax.experimental import pallas as pl
from jax.experimental.pallas import tpu as pltpu
from jax.experimental.pallas import tpu_sc as plsc

N = 10000
E = 320000
D_IN = 128
D_H = 128
D_Z = 64

NC = 2
NS = 16
NW = NC * NS
L = 16

CH = 128
NCHUNK = E // CH
CHUNKS_LO = NCHUNK // NW
CHUNKS_REM = NCHUNK % NW
ROWS_PER_TILE = N // NS

_MESH = plsc.VectorSubcoreMesh(core_axis_name="c", subcore_axis_name="s")


def _wid():
    return lax.axis_index("c") * NS + lax.axis_index("s")


def _nchunks(wid):
    return jnp.where(wid < CHUNKS_REM, CHUNKS_LO + 1, CHUNKS_LO)


@functools.partial(
    pl.kernel,
    out_type=jax.ShapeDtypeStruct((NC, N, L), jnp.float32),
    mesh=_MESH,
    scratch_types=[
        pltpu.VMEM((CH,), jnp.int32),
        pltpu.VMEM((CH, L), jnp.float32),
        pltpu.VMEM((CH, L), jnp.float32),
        pltpu.VMEM_SHARED((N, L), jnp.float32),
    ],
)
def _deg_sc(dst_hbm, deg_hbm, idx_v, ones_v, zb_v, acc):
    cid = lax.axis_index("c")
    sid = lax.axis_index("s")
    wid = _wid()

    def fill(r, _):
        ones_v[r, :] = jnp.full((L,), 1.0, jnp.float32)
        zb_v[r, :] = jnp.zeros((L,), jnp.float32)
        return 0

    lax.fori_loop(0, CH, fill, 0)
    for k in range(5):
        pltpu.sync_copy(
            zb_v.at[pl.ds(0, 125)],
            acc.at[pl.ds(sid * ROWS_PER_TILE + k * 125, 125)],
        )
    plsc.subcore_barrier()

    def body(c, _):
        base = (c * NW + wid) * CH
        pltpu.sync_copy(dst_hbm.at[pl.ds(base, CH)], idx_v)
        pltpu.sync_copy(ones_v, acc.at[idx_v], add=True)
        return 0

    lax.fori_loop(0, _nchunks(wid), body, 0)
    plsc.subcore_barrier()
    pltpu.sync_copy(
        acc.at[pl.ds(sid * ROWS_PER_TILE, ROWS_PER_TILE)],
        deg_hbm.at[cid, pl.ds(sid * ROWS_PER_TILE, ROWS_PER_TILE)],
    )


@functools.partial(
    pl.kernel,
    out_type=jax.ShapeDtypeStruct((NC, N, D_H), jnp.float32),
    mesh=_MESH,
    scratch_types=[
        pltpu.VMEM((CH,), jnp.int32),
        pltpu.VMEM((CH,), jnp.int32),
        pltpu.VMEM((CH, D_H), jnp.float32),
        pltpu.VMEM((CH, D_H), jnp.float32),
        pltpu.VMEM_SHARED((N, D_H), jnp.float32),
        pltpu.SemaphoreType.DMA,
    ],
)
def _agg_sc(g_hbm, src_hbm, dst_hbm, out_hbm, idx_s, idx_d, rows_v, zb_v, acc, sem):
    cid = lax.axis_index("c")
    sid = lax.axis_index("s")
    wid = _wid()

    def fill(r, _):
        for c8 in range(D_H // L):
            zb_v[r, pl.ds(c8 * L, L)] = jnp.zeros((L,), jnp.float32)
        return 0

    lax.fori_loop(0, CH, fill, 0)
    for k in range(5):
        pltpu.sync_copy(
            zb_v.at[pl.ds(0, 125)],
            acc.at[pl.ds(sid * ROWS_PER_TILE + k * 125, 125)],
        )
    plsc.subcore_barrier()

    def body(c, _):
        base = (c * NW + wid) * CH
        pltpu.sync_copy(src_hbm.at[pl.ds(base, CH)], idx_s)
        pltpu.sync_copy(dst_hbm.at[pl.ds(base, CH)], idx_d)
        pltpu.async_copy(g_hbm.at[idx_s], rows_v, sem).wait()
        pltpu.sync_copy(rows_v, acc.at[idx_d], add=True)
        return 0

    lax.fori_loop(0, _nchunks(wid), body, 0)
    plsc.subcore_barrier()
    pltpu.sync_copy(
        acc.at[pl.ds(sid * ROWS_PER_TILE, ROWS_PER_TILE)],
        out_hbm.at[cid, pl.ds(sid * ROWS_PER_TILE, ROWS_PER_TILE)],
    )


@functools.partial(
    pl.kernel,
    out_type=jax.ShapeDtypeStruct((E,), jnp.float32),
    mesh=_MESH,
    scratch_types=[
        pltpu.VMEM((CH,), jnp.int32),
        pltpu.VMEM((CH,), jnp.int32),
        pltpu.VMEM((CH, D_Z), jnp.float32),
        pltpu.VMEM((CH, D_Z), jnp.float32),
        pltpu.VMEM((CH * L,), jnp.float32),
        pltpu.VMEM((CH,), jnp.float32),
        pltpu.SemaphoreType.DMA,
    ],
)
def _dec_sc(z_hbm, src_hbm, dst_hbm, out_hbm, idx_s, idx_d, zs_v, zd_v, q_v,
            o_v, sem):
    wid = _wid()
    lanes = lax.iota(jnp.int32, (L,))

    def body(c, _):
        base = (c * NW + wid) * CH
        pltpu.sync_copy(src_hbm.at[pl.ds(base, CH)], idx_s)
        pltpu.sync_copy(dst_hbm.at[pl.ds(base, CH)], idx_d)
        pltpu.async_copy(z_hbm.at[idx_s], zs_v, sem).wait()
        pltpu.async_copy(z_hbm.at[idx_d], zd_v, sem).wait()

        def dot_edge(e, _):
            q = zs_v[e, pl.ds(0, L)] * zd_v[e, pl.ds(0, L)]
            for s in range(1, D_Z // L):
                q = q + zs_v[e, pl.ds(s * L, L)] * zd_v[e, pl.ds(s * L, L)]
            q_v[pl.ds(e * L, L)] = q
            return 0

        lax.fori_loop(0, CH, dot_edge, 0)

        for grp in range(CH // L):
            r = plsc.load_gather(q_v, [lanes * L + grp * (L * L)])
            for j in range(1, L):
                r = r + plsc.load_gather(q_v, [lanes * L + (grp * (L * L) + j)])
            o_v[pl.ds(grp * L, L)] = 1.0 / (1.0 + jnp.exp(-r))
        pltpu.sync_copy(o_v, out_hbm.at[pl.ds(base, CH)])
        return 0

    lax.fori_loop(0, _nchunks(wid), body, 0)


def _t1_body(x_ref, w_ref, d0_ref, d1_ref, g_ref):
    deg = d0_ref[:, 0:1] + d1_ref[:, 0:1] + 1.0
    dinv = lax.rsqrt(jnp.maximum(deg, 1e-12))
    h = jnp.dot(x_ref[...], w_ref[...], preferred_element_type=jnp.float32)
    g_ref[...] = h * dinv


def _t2_body(s0_ref, s1_ref, g1_ref, d0_ref, d1_ref, b1_ref, w_ref, g2_ref):
    deg = d0_ref[:, 0:1] + d1_ref[:, 0:1] + 1.0
    dinv = lax.rsqrt(jnp.maximum(deg, 1e-12))
    h = jnp.maximum(
        dinv * (s0_ref[...] + s1_ref[...] + g1_ref[...]) + b1_ref[...], 0.0)
    p = jnp.dot(h, w_ref[...], preferred_element_type=jnp.float32)
    g2_ref[...] = p * dinv


def _t3_body(s0_ref, s1_ref, g2_ref, d0_ref, d1_ref, bc_ref, eps_ref, z_ref):
    deg = d0_ref[:, 0:1] + d1_ref[:, 0:1] + 1.0
    dinv = lax.rsqrt(jnp.maximum(deg, 1e-12))
    o = dinv * (s0_ref[...] + s1_ref[...] + g2_ref[...]) + bc_ref[...]
    mu = o[:, :D_Z]
    lv = o[:, D_Z:]
    z_ref[...] = mu + jnp.exp(0.5 * lv) * eps_ref[...]


_RB = 500
_GRID = N // _RB


def _row_spec(width):
    return pl.BlockSpec((_RB, width), lambda i: (i, 0))


def _full_spec(shape):
    return pl.BlockSpec(shape, lambda i: tuple(0 for _ in shape))


def kernel(x, edge_index, W1, b1, W_mu, b_mu, W_lv, b_lv):
    src = edge_index[0]
    dst = edge_index[1]
    Wcat = jnp.concatenate([W_mu, W_lv], axis=1)
    bcat = jnp.concatenate([b_mu, b_lv], axis=0).reshape(1, 2 * D_Z)
    b1r = b1.reshape(1, D_H)
    eps = jax.random.normal(jax.random.key(42), (N, D_Z), jnp.float32)

    deg_parts = _deg_sc(dst)
    d0 = deg_parts[0]
    d1 = deg_parts[1]

    g1 = pl.pallas_call(
        _t1_body,
        grid=(_GRID,),
        in_specs=[_row_spec(D_IN), _full_spec((D_IN, D_H)), _row_spec(L),
                  _row_spec(L)],
        out_specs=_row_spec(D_H),
        out_shape=jax.ShapeDtypeStruct((N, D_H), jnp.float32),
    )(x, W1, d0, d1)

    s1 = _agg_sc(g1, src, dst)

    g2 = pl.pallas_call(
        _t2_body,
        grid=(_GRID,),
        in_specs=[_row_spec(D_H), _row_spec(D_H), _row_spec(D_H),
                  _row_spec(L), _row_spec(L), _full_spec((1, D_H)),
                  _full_spec((D_H, D_H))],
        out_specs=_row_spec(D_H),
        out_shape=jax.ShapeDtypeStruct((N, D_H), jnp.float32),
    )(s1[0], s1[1], g1, d0, d1, b1r, Wcat)

    s2 = _agg_sc(g2, src, dst)

    z = pl.pallas_call(
        _t3_body,
        grid=(_GRID,),
        in_specs=[_row_spec(D_H), _row_spec(D_H), _row_spec(D_H),
                  _row_spec(L), _row_spec(L), _full_spec((1, D_H)),
                  _row_spec(D_Z)],
        out_specs=_row_spec(D_Z),
        out_shape=jax.ShapeDtypeStruct((N, D_Z), jnp.float32),
    )(s2[0], s2[1], g2, d0, d1, bcat, eps)

    return _dec_sc(z, src, dst)


# speedup vs baseline: 10.3366x; 10.3366x over previous
import functools

import jax
import jax.numpy as jnp
from jax import lax
from jax.experimental import pallas as pl
from jax.experimental.pallas import tpu as pltpu
from jax.experimental.pallas import tpu_sc as plsc

N = 10000
E = 320000
D_IN = 128
D_H = 128
D_Z = 64

NC = 2
NS = 16
NW = NC * NS
L = 16

CH = 128
NCHUNK = E // CH
CHUNKS_LO = NCHUNK // NW
CHUNKS_REM = NCHUNK % NW
NPAD = 10240
ROWS_PER_TILE = NPAD // NS

_MESH = plsc.VectorSubcoreMesh(core_axis_name="c", subcore_axis_name="s", num_cores=2, num_subcores=16)


def _wid():
    return lax.axis_index("c") * NS + lax.axis_index("s")


def _nchunks(wid):
    return jnp.where(wid < CHUNKS_REM, CHUNKS_LO + 1, CHUNKS_LO)


@functools.partial(
    pl.kernel,
    out_type=jax.ShapeDtypeStruct((NC, NPAD, L), jnp.float32),
    mesh=_MESH,
    scratch_types=[
        pltpu.VMEM((CH,), jnp.int32),
        pltpu.VMEM((CH, L), jnp.float32),
        pltpu.VMEM((CH, L), jnp.float32),
        pltpu.VMEM_SHARED((NPAD, L), jnp.float32),
    ],
)
def _deg_sc(dst_hbm, deg_hbm, idx_v, ones_v, zb_v, acc):
    cid = lax.axis_index("c")
    sid = lax.axis_index("s")
    wid = _wid()

    def fill(r, _):
        ones_v[r, :] = jnp.full((L,), 1.0, jnp.float32)
        zb_v[r, :] = jnp.zeros((L,), jnp.float32)
        return 0

    lax.fori_loop(0, CH, fill, 0)
    for k in range(5):
        pltpu.sync_copy(
            zb_v, acc.at[pl.ds(sid * ROWS_PER_TILE + k * CH, CH)])
    plsc.subcore_barrier()

    def body(c, _):
        base = (c * NW + wid) * CH
        pltpu.sync_copy(dst_hbm.at[pl.ds(base, CH)], idx_v)
        pltpu.sync_copy(ones_v, acc.at[idx_v], add=True)
        return 0

    lax.fori_loop(0, _nchunks(wid), body, 0)
    plsc.subcore_barrier()
    pltpu.sync_copy(
        acc.at[pl.ds(sid * ROWS_PER_TILE, ROWS_PER_TILE)],
        deg_hbm.at[cid, pl.ds(sid * ROWS_PER_TILE, ROWS_PER_TILE)],
    )


@functools.partial(
    pl.kernel,
    out_type=jax.ShapeDtypeStruct((NC, NPAD, D_H), jnp.float32),
    mesh=_MESH,
    scratch_types=[
        pltpu.VMEM((CH,), jnp.int32),
        pltpu.VMEM((CH,), jnp.int32),
        pltpu.VMEM((CH, D_H), jnp.float32),
        pltpu.VMEM((CH, D_H), jnp.float32),
        pltpu.VMEM_SHARED((NPAD, D_H), jnp.float32),
        pltpu.SemaphoreType.DMA,
    ],
)
def _agg_sc(g_hbm, src_hbm, dst_hbm, out_hbm, idx_s, idx_d, rows_v, zb_v, acc, sem):
    cid = lax.axis_index("c")
    sid = lax.axis_index("s")
    wid = _wid()

    def fill(r, _):
        for c8 in range(D_H // L):
            zb_v[r, pl.ds(c8 * L, L)] = jnp.zeros((L,), jnp.float32)
        return 0

    lax.fori_loop(0, CH, fill, 0)
    for k in range(5):
        pltpu.sync_copy(
            zb_v, acc.at[pl.ds(sid * ROWS_PER_TILE + k * CH, CH)])
    plsc.subcore_barrier()

    def body(c, _):
        base = (c * NW + wid) * CH
        pltpu.sync_copy(src_hbm.at[pl.ds(base, CH)], idx_s)
        pltpu.sync_copy(dst_hbm.at[pl.ds(base, CH)], idx_d)
        pltpu.async_copy(g_hbm.at[idx_s], rows_v, sem).wait()
        pltpu.sync_copy(rows_v, acc.at[idx_d], add=True)
        return 0

    lax.fori_loop(0, _nchunks(wid), body, 0)
    plsc.subcore_barrier()
    pltpu.sync_copy(
        acc.at[pl.ds(sid * ROWS_PER_TILE, ROWS_PER_TILE)],
        out_hbm.at[cid, pl.ds(sid * ROWS_PER_TILE, ROWS_PER_TILE)],
    )


@functools.partial(
    pl.kernel,
    out_type=jax.ShapeDtypeStruct((E * L,), jnp.float32),
    mesh=_MESH,
    scratch_types=[
        pltpu.VMEM((CH,), jnp.int32),
        pltpu.VMEM((CH,), jnp.int32),
        pltpu.VMEM((CH, D_H), jnp.float32),
        pltpu.VMEM((CH, D_H), jnp.float32),
        pltpu.VMEM((CH * L,), jnp.float32),
        pltpu.SemaphoreType.DMA,
    ],
)
def _dec_sc(z_hbm, src_hbm, dst_hbm, q_hbm, idx_s, idx_d, zs_v, zd_v, q_v,
            sem):
    wid = _wid()

    def body(c, _):
        base = (c * NW + wid) * CH
        pltpu.sync_copy(src_hbm.at[pl.ds(base, CH)], idx_s)
        pltpu.sync_copy(dst_hbm.at[pl.ds(base, CH)], idx_d)
        pltpu.async_copy(z_hbm.at[idx_s], zs_v, sem).wait()
        pltpu.async_copy(z_hbm.at[idx_d], zd_v, sem).wait()

        def dot_edge(e, _):
            q = zs_v[e, pl.ds(0, L)] * zd_v[e, pl.ds(0, L)]
            for s in range(1, D_Z // L):
                q = q + zs_v[e, pl.ds(s * L, L)] * zd_v[e, pl.ds(s * L, L)]
            q_v[pl.ds(e * L, L)] = q
            return 0

        lax.fori_loop(0, CH, dot_edge, 0)
        pltpu.sync_copy(q_v, q_hbm.at[pl.ds(base * L, CH * L)])
        return 0

    lax.fori_loop(0, _nchunks(wid), body, 0)


def _t1_body(x_ref, w_ref, d0_ref, d1_ref, g_ref):
    deg = d0_ref[:, 0:1] + d1_ref[:, 0:1] + 1.0
    dinv = lax.rsqrt(jnp.maximum(deg, 1e-12))
    h = jnp.dot(x_ref[...], w_ref[...], preferred_element_type=jnp.float32)
    g_ref[...] = h * dinv


def _t2_body(s0_ref, s1_ref, g1_ref, d0_ref, d1_ref, b1_ref, w_ref, g2_ref):
    deg = d0_ref[:, 0:1] + d1_ref[:, 0:1] + 1.0
    dinv = lax.rsqrt(jnp.maximum(deg, 1e-12))
    h = jnp.maximum(
        dinv * (s0_ref[...] + s1_ref[...] + g1_ref[...]) + b1_ref[...], 0.0)
    p = jnp.dot(h, w_ref[...], preferred_element_type=jnp.float32)
    g2_ref[...] = p * dinv


def _t4_body(q_ref, o_ref):
    o_ref[...] = jax.nn.sigmoid(jnp.sum(q_ref[...], axis=1, keepdims=True))


def _t3_body(s0_ref, s1_ref, g2_ref, d0_ref, d1_ref, bc_ref, eps_ref, z_ref):
    deg = d0_ref[:, 0:1] + d1_ref[:, 0:1] + 1.0
    dinv = lax.rsqrt(jnp.maximum(deg, 1e-12))
    o = dinv * (s0_ref[...] + s1_ref[...] + g2_ref[...]) + bc_ref[...]
    mu = o[:, :D_Z]
    lv = o[:, D_Z:]
    z = mu + jnp.exp(0.5 * lv) * eps_ref[...]
    z_ref[...] = jnp.concatenate([z, jnp.zeros_like(z)], axis=1)


_RB = 1000
_GRID = N // _RB


def _row_spec(width):
    return pl.BlockSpec((_RB, width), lambda i: (i, 0))


def _full_spec(shape):
    return pl.BlockSpec(shape, lambda i: tuple(0 for _ in shape))


def kernel(x, edge_index, W1, b1, W_mu, b_mu, W_lv, b_lv):
    src = edge_index[0]
    dst = edge_index[1]
    Wcat = jnp.concatenate([W_mu, W_lv], axis=1)
    bcat = jnp.concatenate([b_mu, b_lv], axis=0).reshape(1, 2 * D_Z)
    b1r = b1.reshape(1, D_H)
    eps = jax.random.normal(jax.random.key(42), (N, D_Z), jnp.float32)

    deg_parts = _deg_sc(dst)
    d0 = deg_parts[0, :N]
    d1 = deg_parts[1, :N]

    g1 = pl.pallas_call(
        _t1_body,
        grid=(_GRID,),
        in_specs=[_row_spec(D_IN), _full_spec((D_IN, D_H)), _row_spec(L),
                  _row_spec(L)],
        out_specs=_row_spec(D_H),
        out_shape=jax.ShapeDtypeStruct((N, D_H), jnp.float32),
    )(x, W1, d0, d1)

    s1p = _agg_sc(g1, src, dst)
    s1 = (s1p[0, :N], s1p[1, :N])

    g2 = pl.pallas_call(
        _t2_body,
        grid=(_GRID,),
        in_specs=[_row_spec(D_H), _row_spec(D_H), _row_spec(D_H),
                  _row_spec(L), _row_spec(L), _full_spec((1, D_H)),
                  _full_spec((D_H, D_H))],
        out_specs=_row_spec(D_H),
        out_shape=jax.ShapeDtypeStruct((N, D_H), jnp.float32),
    )(s1[0], s1[1], g1, d0, d1, b1r, Wcat)

    s2p = _agg_sc(g2, src, dst)
    s2 = (s2p[0, :N], s2p[1, :N])

    z = pl.pallas_call(
        _t3_body,
        grid=(_GRID,),
        in_specs=[_row_spec(D_H), _row_spec(D_H), _row_spec(D_H),
                  _row_spec(L), _row_spec(L), _full_spec((1, D_H)),
                  _row_spec(D_Z)],
        out_specs=_row_spec(D_H),
        out_shape=jax.ShapeDtypeStruct((N, D_H), jnp.float32),
    )(s2[0], s2[1], g2, d0, d1, bcat, eps)

    qflat = _dec_sc(z, src, dst)
    q = qflat.reshape(E, L)

    _EB = 4000
    out = pl.pallas_call(
        _t4_body,
        grid=(E // _EB,),
        in_specs=[pl.BlockSpec((_EB, L), lambda i: (i, 0))],
        out_specs=pl.BlockSpec((_EB, 1), lambda i: (i, 0)),
        out_shape=jax.ShapeDtypeStruct((E, 1), jnp.float32),
    )(q)
    return out.reshape(E)
